# baseline (speedup 1.0000x reference)
import functools

import jax
import jax.numpy as jnp
import numpy as np
from jax import lax
from jax.experimental import pallas as pl
from jax.experimental.pallas import tpu as pltpu
from jax.experimental.pallas import tpu_sc as plsc

D = 512
S = 256
B = 1024
NW = 32
PPW = S // NW
NBB = 8
RPC = NBB * PPW
NCH = B // NBB
NBUF = 3
L = 16


def _positional_encoding_np():
    positions = np.arange(S)[:, np.newaxis]
    rates = 1 / 10000 ** (np.arange(0, D, 2)[np.newaxis, :] / D)
    radians = positions * rates
    return np.concatenate([np.sin(radians), np.cos(radians)], axis=-1).astype(np.float32)


_PE = _positional_encoding_np()


def _make_sc_kernel():
    mesh = plsc.VectorSubcoreMesh(core_axis_name="c", subcore_axis_name="s")

    @functools.partial(
        pl.kernel,
        mesh=mesh,
        out_type=jax.ShapeDtypeStruct((B, S, D), jnp.float32),
        scratch_types=[
            pltpu.VMEM((B * PPW,), jnp.int32),
            pltpu.VMEM((PPW, D), jnp.float32),
            pltpu.VMEM((RPC, D), jnp.float32),
            pltpu.VMEM((RPC, D), jnp.float32),
            pltpu.VMEM((RPC, D), jnp.float32),
            pltpu.SemaphoreType.DMA,
            pltpu.SemaphoreType.DMA,
            pltpu.SemaphoreType.DMA,
            pltpu.SemaphoreType.DMA,
            pltpu.SemaphoreType.DMA,
            pltpu.SemaphoreType.DMA,
        ],
    )
    def emb(enc2_hbm, table_hbm, pe_hbm, out_hbm, idx_all, pe_v,
            rows0, rows1, rows2, gsem0, gsem1, gsem2, wsem0, wsem1, wsem2):
        cid = lax.axis_index("c")
        sid = lax.axis_index("s")
        wid = sid * 2 + cid
        p0 = wid * PPW

        pltpu.sync_copy(enc2_hbm.at[wid], idx_all)
        pltpu.sync_copy(pe_hbm.at[pl.ds(p0, PPW), :], pe_v)

        rows = (rows0, rows1, rows2)
        gsems = (gsem0, gsem1, gsem2)
        wsems = (wsem0, wsem1, wsem2)

        def idx_slice(c):
            return idx_all.at[pl.ds(c * RPC, RPC)]

        def fire_gather(c, q):
            pltpu.async_copy(table_hbm.at[idx_slice(c)], rows[q], gsems[q])

        def wait_gather(c, q):
            pltpu.make_async_copy(table_hbm.at[idx_slice(c)], rows[q], gsems[q]).wait()

        def fire_write(c, q):
            b0 = c * NBB
            for bb in range(NBB):
                pltpu.async_copy(
                    rows[q].at[pl.ds(bb * PPW, PPW), :],
                    out_hbm.at[b0 + bb, pl.ds(p0, PPW), :],
                    wsems[q],
                )

        def wait_write(c, q):
            pltpu.make_async_copy(
                rows[q], out_hbm.at[pl.ds(c * NBB, NBB), pl.ds(p0, PPW), :], wsems[q]
            ).wait()

        def process(c, q):
            wait_gather(c, q)
            for j in range(PPW):
                pe_rows = [pe_v[j, pl.ds(k * L, L)] for k in range(D // L)]

                @plsc.parallel_loop(0, NBB, unroll=2)
                def add_row(b):
                    for k in range(D // L):
                        rows[q][b * PPW + j, pl.ds(k * L, L)] = (
                            rows[q][b * PPW + j, pl.ds(k * L, L)] + pe_rows[k]
                        )

            fire_write(c, q)

        NTRIP = (NCH + 1 + NBUF - 1) // NBUF

        def trip(c3, _):
            for q in range(NBUF):
                c = c3 * NBUF + q

                @pl.when(jnp.logical_and(c >= NBUF, c < NCH))
                def _():
                    wait_write(c - NBUF, q)

                @pl.when(c < NCH)
                def _():
                    fire_gather(c, q)

                @pl.when(jnp.logical_and(c >= 1, c <= NCH))
                def _():
                    process(c - 1, (q - 1) % NBUF)

            return 0

        lax.fori_loop(0, NTRIP, trip, 0)

        for cc in range(NCH - NBUF, NCH):
            wait_write(cc, cc % NBUF)

    return emb


_emb = _make_sc_kernel()


def kernel(encoding, table):
    enc2 = (
        encoding.reshape(B, NW, PPW)
        .transpose(1, 0, 2)
        .reshape(NW, B * PPW)
    )
    return _emb(enc2, table, jnp.asarray(_PE))


# speedup vs baseline: 1.1023x; 1.1023x over previous
import functools

import jax
import jax.numpy as jnp
import numpy as np
from jax import lax
from jax.experimental import pallas as pl
from jax.experimental.pallas import tpu as pltpu
from jax.experimental.pallas import tpu_sc as plsc

D = 512
S = 256
B = 1024
NW = 32
PPW = S // NW
NB = 64
CPP = B // NB
NCH = PPW * CPP
NBUF = 3
L = 16


def _positional_encoding_np():
    positions = np.arange(S)[:, np.newaxis]
    rates = 1 / 10000 ** (np.arange(0, D, 2)[np.newaxis, :] / D)
    radians = positions * rates
    return np.concatenate([np.sin(radians), np.cos(radians)], axis=-1).astype(np.float32)


_PE = _positional_encoding_np()


def _make_sc_kernel():
    mesh = plsc.VectorSubcoreMesh(core_axis_name="c", subcore_axis_name="s")

    @functools.partial(
        pl.kernel,
        mesh=mesh,
        out_type=jax.ShapeDtypeStruct((B, S, D), jnp.float32),
        scratch_types=[
            pltpu.VMEM((PPW, B), jnp.int32),
            pltpu.VMEM((PPW, D), jnp.float32),
            pltpu.VMEM((NB, D), jnp.float32),
            pltpu.VMEM((NB, D), jnp.float32),
            pltpu.VMEM((NB, D), jnp.float32),
            pltpu.SemaphoreType.DMA,
            pltpu.SemaphoreType.DMA,
            pltpu.SemaphoreType.DMA,
            pltpu.SemaphoreType.DMA,
            pltpu.SemaphoreType.DMA,
            pltpu.SemaphoreType.DMA,
        ],
    )
    def emb(enc_t_hbm, table_hbm, pe_hbm, out_hbm, idx_all, pe_v,
            rows0, rows1, rows2, gsem0, gsem1, gsem2, wsem0, wsem1, wsem2):
        cid = lax.axis_index("c")
        sid = lax.axis_index("s")
        wid = sid * 2 + cid
        p0 = wid * PPW

        pltpu.sync_copy(enc_t_hbm.at[pl.ds(p0, PPW), :], idx_all)
        pltpu.sync_copy(pe_hbm.at[pl.ds(p0, PPW), :], pe_v)

        rows = (rows0, rows1, rows2)
        gsems = (gsem0, gsem1, gsem2)
        wsems = (wsem0, wsem1, wsem2)

        def chunk_j(c):
            return c // CPP

        def chunk_b0(c):
            return (c % CPP) * NB

        def idx_slice(c):
            return idx_all.at[chunk_j(c), pl.ds(chunk_b0(c), NB)]

        def out_slice(c):
            return out_hbm.at[pl.ds(chunk_b0(c), NB), p0 + chunk_j(c), :]

        def fire_gather(c, q):
            pltpu.async_copy(table_hbm.at[idx_slice(c)], rows[q], gsems[q])

        def wait_gather(c, q):
            pltpu.make_async_copy(table_hbm.at[idx_slice(c)], rows[q], gsems[q]).wait()

        def fire_write(c, q):
            pltpu.async_copy(rows[q], out_slice(c), wsems[q])

        def wait_write(c, q):
            pltpu.make_async_copy(rows[q], out_slice(c), wsems[q]).wait()

        def process(c, q):
            wait_gather(c, q)
            j = chunk_j(c)
            pe_rows = [pe_v[j, pl.ds(k * L, L)] for k in range(D // L)]

            @plsc.parallel_loop(0, NB, unroll=2)
            def add_row(b):
                for k in range(D // L):
                    rows[q][b, pl.ds(k * L, L)] = (
                        rows[q][b, pl.ds(k * L, L)] + pe_rows[k]
                    )

            fire_write(c, q)

        NTRIP = (NCH + 1 + NBUF - 1) // NBUF

        def trip(c3, _):
            for q in range(NBUF):
                c = c3 * NBUF + q

                @pl.when(jnp.logical_and(c >= NBUF, c < NCH))
                def _():
                    wait_write(c - NBUF, q)

                @pl.when(c < NCH)
                def _():
                    fire_gather(c, q)

                @pl.when(jnp.logical_and(c >= 1, c <= NCH))
                def _():
                    process(c - 1, (q - 1) % NBUF)

            return 0

        lax.fori_loop(0, NTRIP, trip, 0)

        for cc in range(NCH - NBUF, NCH):
            wait_write(cc, cc % NBUF)

    return emb


_emb = _make_sc_kernel()


def kernel(encoding, table):
    enc_t = encoding.T
    return _emb(enc_t, table, jnp.asarray(_PE))


# speedup vs baseline: 1.1119x; 1.0087x over previous
import functools

import jax
import jax.numpy as jnp
import numpy as np
from jax import lax
from jax.experimental import pallas as pl
from jax.experimental.pallas import tpu as pltpu
from jax.experimental.pallas import tpu_sc as plsc

D = 512
S = 256
B = 1024
NW = 32
PPW = S // NW
NB = 64
CPP = B // NB
NCH = PPW * CPP
NBUF = 3
L = 16


def _positional_encoding_np():
    positions = np.arange(S)[:, np.newaxis]
    rates = 1 / 10000 ** (np.arange(0, D, 2)[np.newaxis, :] / D)
    radians = positions * rates
    return np.concatenate([np.sin(radians), np.cos(radians)], axis=-1).astype(np.float32)


_PE = _positional_encoding_np()


def _make_sc_kernel():
    mesh = plsc.VectorSubcoreMesh(core_axis_name="c", subcore_axis_name="s")

    @functools.partial(
        pl.kernel,
        mesh=mesh,
        out_type=jax.ShapeDtypeStruct((B, S, D), jnp.float32),
        scratch_types=[
            pltpu.VMEM((PPW, B), jnp.int32),
            pltpu.VMEM((PPW, D), jnp.float32),
            pltpu.VMEM((NB, D), jnp.float32),
            pltpu.VMEM((NB, D), jnp.float32),
            pltpu.VMEM((NB, D), jnp.float32),
            pltpu.SemaphoreType.DMA,
            pltpu.SemaphoreType.DMA,
            pltpu.SemaphoreType.DMA,
            pltpu.SemaphoreType.DMA,
            pltpu.SemaphoreType.DMA,
            pltpu.SemaphoreType.DMA,
        ],
    )
    def emb(enc_t_hbm, table_hbm, pe_hbm, out_hbm, idx_all, pe_v,
            rows0, rows1, rows2, gsem0, gsem1, gsem2, wsem0, wsem1, wsem2):
        cid = lax.axis_index("c")
        sid = lax.axis_index("s")
        wid = sid * 2 + cid
        p0 = wid * PPW

        pltpu.sync_copy(enc_t_hbm.at[pl.ds(p0, PPW), :], idx_all)
        pltpu.sync_copy(pe_hbm.at[pl.ds(p0, PPW), :], pe_v)

        rows = (rows0, rows1, rows2)
        gsems = (gsem0, gsem1, gsem2)
        wsems = (wsem0, wsem1, wsem2)

        def chunk_j(c):
            return c // CPP

        def chunk_b0(c):
            return (c % CPP) * NB

        def idx_slice(c):
            return idx_all.at[chunk_j(c), pl.ds(chunk_b0(c), NB)]

        def out_slice(c):
            return out_hbm.at[pl.ds(chunk_b0(c), NB), p0 + chunk_j(c), :]

        def fire_gather(c, q):
            pltpu.async_copy(table_hbm.at[idx_slice(c)], rows[q], gsems[q])

        def wait_gather(c, q):
            pltpu.make_async_copy(table_hbm.at[idx_slice(c)], rows[q], gsems[q]).wait()

        def fire_write(c, q):
            pltpu.async_copy(rows[q], out_slice(c), wsems[q])

        def wait_write(c, q):
            pltpu.make_async_copy(rows[q], out_slice(c), wsems[q]).wait()

        def process(c, q):
            wait_gather(c, q)
            j = chunk_j(c)
            pe_rows = [pe_v[j, pl.ds(k * L, L)] for k in range(D // L)]

            def add_row(b, _):
                for k in range(D // L):
                    rows[q][b, pl.ds(k * L, L)] = (
                        rows[q][b, pl.ds(k * L, L)] + pe_rows[k]
                    )
                return 0

            lax.fori_loop(0, NB, add_row, 0)
            fire_write(c, q)

        NTRIP = (NCH + 1 + NBUF - 1) // NBUF

        def trip(c3, _):
            for q in range(NBUF):
                c = c3 * NBUF + q

                @pl.when(jnp.logical_and(c >= NBUF, c < NCH))
                def _():
                    wait_write(c - NBUF, q)

                @pl.when(c < NCH)
                def _():
                    fire_gather(c, q)

                @pl.when(jnp.logical_and(c >= 1, c <= NCH))
                def _():
                    process(c - 1, (q - 1) % NBUF)

            return 0

        lax.fori_loop(0, NTRIP, trip, 0)

        for cc in range(NCH - NBUF, NCH):
            wait_write(cc, cc % NBUF)

    return emb


_emb = _make_sc_kernel()


def kernel(encoding, table):
    enc_t = encoding.T
    return _emb(enc_t, table, jnp.asarray(_PE))
